# Initial kernel scaffold; baseline (speedup 1.0000x reference)
import jax
import jax.numpy as jnp
from jax.experimental import pallas as pl


def kernel(xyz, feats, child_xyz):
    raise NotImplementedError("write your pallas kernel here")


# speedup vs baseline: 54.9675x; 54.9675x over previous
import functools

import jax
import jax.numpy as jnp
from jax import lax
from jax.experimental import pallas as pl
from jax.experimental.pallas import tpu as pltpu, tpu_sc as plsc

BS = 2
N = 8192
M = 4096
C = 128
BM = 128
BIG = 1e30

NC = 2
NS = 16
NW = NC * NS
CPW = (BS * M) // NW
CHUNK = 32
ROWS_PER_CHUNK = CHUNK * 3
NCHUNK = CPW // CHUNK


def _knn_body(child_ref, xyz_ref, idx_ref, w_ref):
    b = pl.program_id(0)
    c = child_ref[0]
    p = xyz_ref[0]
    t0 = c[:, 0:1] - p[0:1, :]
    t1 = c[:, 1:2] - p[1:2, :]
    t2 = c[:, 2:3] - p[2:3, :]
    d2 = (t0 * t0 + t1 * t1) + t2 * t2

    iota = lax.broadcasted_iota(jnp.int32, (BM, N), 1)

    m1 = jnp.min(d2, axis=1, keepdims=True)
    a1 = jnp.min(jnp.where(d2 == m1, iota, N), axis=1, keepdims=True)
    d2b = jnp.where(iota == a1, BIG, d2)
    m2 = jnp.min(d2b, axis=1, keepdims=True)
    a2 = jnp.min(jnp.where(d2b == m2, iota, N), axis=1, keepdims=True)
    d2c = jnp.where(iota == a2, BIG, d2b)
    m3 = jnp.min(d2c, axis=1, keepdims=True)
    a3 = jnp.min(jnp.where(d2c == m3, iota, N), axis=1, keepdims=True)

    d = jnp.sqrt(jnp.maximum(jnp.concatenate([m1, m2, m3], axis=1), 1e-12))
    inv = 1.0 / (d + 1e-8)
    w = inv / jnp.sum(inv, axis=1, keepdims=True)

    idx_ref[0] = jnp.concatenate([a1, a2, a3], axis=1) + b * N
    w_ref[0] = w


def _three_nn_tc(child_xyz, xyz_t):
    grid = (BS, M // BM)
    return pl.pallas_call(
        _knn_body,
        grid=grid,
        in_specs=[
            pl.BlockSpec((1, BM, 3), lambda b, i: (b, i, 0)),
            pl.BlockSpec((1, 3, N), lambda b, i: (b, 0, 0)),
        ],
        out_specs=[
            pl.BlockSpec((1, BM, 3), lambda b, i: (b, i, 0)),
            pl.BlockSpec((1, BM, 3), lambda b, i: (b, i, 0)),
        ],
        out_shape=[
            jax.ShapeDtypeStruct((BS, M, 3), jnp.int32),
            jax.ShapeDtypeStruct((BS, M, 3), jnp.float32),
        ],
    )(child_xyz, xyz_t)


def _gather_body(feats_hbm, idx_hbm, w_hbm, out_hbm,
                 idx_v, w_v, rows_v, out_v, sem):
    wid = lax.axis_index("s") * NC + lax.axis_index("c")
    base_child = wid * CPW
    base_row = base_child * 3

    pltpu.sync_copy(idx_hbm.at[pl.ds(base_row, CPW * 3)], idx_v)
    pltpu.sync_copy(w_hbm.at[pl.ds(base_row, CPW * 3)], w_v.at[pl.ds(0, CPW * 3)])

    def chunk_body(ch, _):
        idx_slice = idx_v.at[pl.ds(ch * ROWS_PER_CHUNK, ROWS_PER_CHUNK)]
        pltpu.async_copy(feats_hbm.at[idx_slice], rows_v, sem).wait()

        def child_body(lc, _):
            r0 = lc * 3
            woff = ch * ROWS_PER_CHUNK + r0
            wv = w_v[pl.ds(woff, 16)]
            w0 = jnp.full((16,), wv[0], jnp.float32)
            w1 = jnp.full((16,), wv[1], jnp.float32)
            w2 = jnp.full((16,), wv[2], jnp.float32)
            oc = ch * CHUNK + lc
            for dc in range(C // 16):
                sl = pl.ds(dc * 16, 16)
                acc = (w0 * rows_v[r0, sl] + w1 * rows_v[r0 + 1, sl]
                       + w2 * rows_v[r0 + 2, sl])
                out_v[oc, sl] = acc
            return ()

        lax.fori_loop(0, CHUNK, child_body, (), unroll=False)
        return ()

    lax.fori_loop(0, NCHUNK, chunk_body, (), unroll=False)
    pltpu.sync_copy(out_v, out_hbm.at[pl.ds(base_child, CPW)])


def _gather_sc(feats_flat, idx_flat, w_flat):
    mesh = plsc.VectorSubcoreMesh(core_axis_name="c", subcore_axis_name="s",
                                  num_cores=NC, num_subcores=NS)
    f = pl.kernel(
        _gather_body,
        out_type=jax.ShapeDtypeStruct((BS * M, C), jnp.float32),
        mesh=mesh,
        scratch_types=[
            pltpu.VMEM((CPW * 3,), jnp.int32),
            pltpu.VMEM((CPW * 3 + 16,), jnp.float32),
            pltpu.VMEM((ROWS_PER_CHUNK, C), jnp.float32),
            pltpu.VMEM((CPW, C), jnp.float32),
            pltpu.SemaphoreType.DMA,
        ],
    )
    return f(feats_flat, idx_flat, w_flat)


@jax.jit
def kernel(xyz, feats, child_xyz):
    xyz_t = jnp.transpose(xyz, (0, 2, 1))
    idx, w = _three_nn_tc(child_xyz, xyz_t)
    feats_flat = jnp.transpose(feats, (0, 2, 1)).reshape(BS * N, C)
    out = _gather_sc(feats_flat, idx.reshape(-1), w.reshape(-1))
    child_feats = jnp.transpose(out.reshape(BS, M, C), (0, 2, 1))
    return (child_xyz, child_feats)


# speedup vs baseline: 64.2837x; 1.1695x over previous
import functools

import jax
import jax.numpy as jnp
from jax import lax
from jax.experimental import pallas as pl
from jax.experimental.pallas import tpu as pltpu, tpu_sc as plsc

BS = 2
N = 8192
M = 4096
C = 128
BM = 256
BIG = 1e30

NC = 2
NS = 16
NW = NC * NS
CPW = (BS * M) // NW
CHUNK = 32
ROWS_PER_CHUNK = CHUNK * 3
NCHUNK = CPW // CHUNK


def _knn_body(child_ref, xyz_ref, idx_ref, w_ref):
    b = pl.program_id(0)
    c = child_ref[0]
    p = xyz_ref[0]
    t0 = c[:, 0:1] - p[0:1, :]
    t1 = c[:, 1:2] - p[1:2, :]
    t2 = c[:, 2:3] - p[2:3, :]
    d2 = (t0 * t0 + t1 * t1) + t2 * t2

    iota = lax.broadcasted_iota(jnp.int32, (BM, N), 1).astype(jnp.float32)
    fn = jnp.float32(N)

    m1 = jnp.min(d2, axis=1, keepdims=True)
    a1 = jnp.min(jnp.where(d2 == m1, iota, fn), axis=1, keepdims=True)
    d2b = jnp.where(iota == a1, BIG, d2)
    m2 = jnp.min(d2b, axis=1, keepdims=True)
    a2 = jnp.min(jnp.where(d2b == m2, iota, fn), axis=1, keepdims=True)
    d2c = jnp.where(iota == a2, BIG, d2b)
    m3 = jnp.min(d2c, axis=1, keepdims=True)
    a3 = jnp.min(jnp.where(d2c == m3, iota, fn), axis=1, keepdims=True)

    d = jnp.sqrt(jnp.maximum(jnp.concatenate([m1, m2, m3], axis=1), 1e-12))
    inv = 1.0 / (d + 1e-8)
    w = inv / jnp.sum(inv, axis=1, keepdims=True)

    ai = jnp.concatenate([a1, a2, a3], axis=1).astype(jnp.int32)
    idx_ref[0] = ai + b * N
    w_ref[0] = w


def _three_nn_tc(child_xyz, xyz_t):
    grid = (BS, M // BM)
    return pl.pallas_call(
        _knn_body,
        grid=grid,
        in_specs=[
            pl.BlockSpec((1, BM, 3), lambda b, i: (b, i, 0)),
            pl.BlockSpec((1, 3, N), lambda b, i: (b, 0, 0)),
        ],
        out_specs=[
            pl.BlockSpec((1, BM, 3), lambda b, i: (b, i, 0)),
            pl.BlockSpec((1, BM, 3), lambda b, i: (b, i, 0)),
        ],
        out_shape=[
            jax.ShapeDtypeStruct((BS, M, 3), jnp.int32),
            jax.ShapeDtypeStruct((BS, M, 3), jnp.float32),
        ],
    )(child_xyz, xyz_t)


def _gather_body(feats_hbm, idx_hbm, w_hbm, out_hbm,
                 idx_v, w_v, rows_v, out_v, sem):
    wid = lax.axis_index("s") * NC + lax.axis_index("c")
    base_child = wid * CPW
    base_row = base_child * 3

    pltpu.sync_copy(idx_hbm.at[pl.ds(base_row, CPW * 3)], idx_v)
    pltpu.sync_copy(w_hbm.at[pl.ds(base_row, CPW * 3)], w_v.at[pl.ds(0, CPW * 3)])

    def chunk_body(ch, _):
        idx_slice = idx_v.at[pl.ds(ch * ROWS_PER_CHUNK, ROWS_PER_CHUNK)]
        pltpu.async_copy(feats_hbm.at[idx_slice], rows_v, sem).wait()

        def child_body(lc, _):
            r0 = lc * 3
            woff = ch * ROWS_PER_CHUNK + r0
            wv = w_v[pl.ds(woff, 16)]
            w0 = jnp.full((16,), wv[0], jnp.float32)
            w1 = jnp.full((16,), wv[1], jnp.float32)
            w2 = jnp.full((16,), wv[2], jnp.float32)
            oc = ch * CHUNK + lc
            for dc in range(C // 16):
                sl = pl.ds(dc * 16, 16)
                acc = (w0 * rows_v[r0, sl] + w1 * rows_v[r0 + 1, sl]
                       + w2 * rows_v[r0 + 2, sl])
                out_v[oc, sl] = acc
            return ()

        lax.fori_loop(0, CHUNK, child_body, (), unroll=False)
        return ()

    lax.fori_loop(0, NCHUNK, chunk_body, (), unroll=False)
    pltpu.sync_copy(out_v, out_hbm.at[pl.ds(base_child, CPW)])


def _gather_sc(feats_flat, idx_flat, w_flat):
    mesh = plsc.VectorSubcoreMesh(core_axis_name="c", subcore_axis_name="s",
                                  num_cores=NC, num_subcores=NS)
    f = pl.kernel(
        _gather_body,
        out_type=jax.ShapeDtypeStruct((BS * M, C), jnp.float32),
        mesh=mesh,
        scratch_types=[
            pltpu.VMEM((CPW * 3,), jnp.int32),
            pltpu.VMEM((CPW * 3 + 16,), jnp.float32),
            pltpu.VMEM((ROWS_PER_CHUNK, C), jnp.float32),
            pltpu.VMEM((CPW, C), jnp.float32),
            pltpu.SemaphoreType.DMA,
        ],
    )
    return f(feats_flat, idx_flat, w_flat)


@jax.jit
def kernel(xyz, feats, child_xyz):
    xyz_t = jnp.transpose(xyz, (0, 2, 1))
    idx, w = _three_nn_tc(child_xyz, xyz_t)
    feats_flat = jnp.transpose(feats, (0, 2, 1)).reshape(BS * N, C)
    out = _gather_sc(feats_flat, idx.reshape(-1), w.reshape(-1))
    child_feats = jnp.transpose(out.reshape(BS, M, C), (0, 2, 1))
    return (child_xyz, child_feats)


# speedup vs baseline: 65.2765x; 1.0154x over previous
import functools

import jax
import jax.numpy as jnp
from jax import lax
from jax.experimental import pallas as pl
from jax.experimental.pallas import tpu as pltpu, tpu_sc as plsc

BS = 2
N = 8192
M = 4096
C = 128
BM = 256
BIG = 1e30

NC = 2
NS = 16
NW = NC * NS
CPW = M // NW
CHUNK = 32
ROWS_PER_CHUNK = CHUNK * 3
NCHUNK = CPW // CHUNK


def _knn_body(b, child_ref, xyz_ref, idx_ref, w_ref):
    c = child_ref[0]
    p = xyz_ref[0]
    t0 = c[:, 0:1] - p[0:1, :]
    t1 = c[:, 1:2] - p[1:2, :]
    t2 = c[:, 2:3] - p[2:3, :]
    d2 = (t0 * t0 + t1 * t1) + t2 * t2

    iota = lax.broadcasted_iota(jnp.int32, (BM, N), 1).astype(jnp.float32)
    fn = jnp.float32(N)

    m1 = jnp.min(d2, axis=1, keepdims=True)
    a1 = jnp.min(jnp.where(d2 == m1, iota, fn), axis=1, keepdims=True)
    d2b = jnp.where(iota == a1, BIG, d2)
    m2 = jnp.min(d2b, axis=1, keepdims=True)
    a2 = jnp.min(jnp.where(d2b == m2, iota, fn), axis=1, keepdims=True)
    d2c = jnp.where(iota == a2, BIG, d2b)
    m3 = jnp.min(d2c, axis=1, keepdims=True)
    a3 = jnp.min(jnp.where(d2c == m3, iota, fn), axis=1, keepdims=True)

    d = jnp.sqrt(jnp.maximum(jnp.concatenate([m1, m2, m3], axis=1), 1e-12))
    inv = 1.0 / (d + 1e-8)
    w = inv / jnp.sum(inv, axis=1, keepdims=True)

    ai = jnp.concatenate([a1, a2, a3], axis=1).astype(jnp.int32)
    idx_ref[0] = ai + b * N
    w_ref[0] = w


def _three_nn_tc(child_xyz, xyz_t, b):
    grid = (M // BM,)
    return pl.pallas_call(
        functools.partial(_knn_body, b),
        grid=grid,
        in_specs=[
            pl.BlockSpec((1, BM, 3), lambda i: (b, i, 0)),
            pl.BlockSpec((1, 3, N), lambda i: (b, 0, 0)),
        ],
        out_specs=[
            pl.BlockSpec((1, BM, 3), lambda i: (0, i, 0)),
            pl.BlockSpec((1, BM, 3), lambda i: (0, i, 0)),
        ],
        out_shape=[
            jax.ShapeDtypeStruct((1, M, 3), jnp.int32),
            jax.ShapeDtypeStruct((1, M, 3), jnp.float32),
        ],
    )(child_xyz, xyz_t)


def _gather_body(feats_hbm, idx_hbm, w_hbm, out_hbm,
                 idx_v, w_v, rows_v, out_v, sem):
    wid = lax.axis_index("s") * NC + lax.axis_index("c")
    base_child = wid * CPW
    base_row = base_child * 3

    pltpu.sync_copy(idx_hbm.at[pl.ds(base_row, CPW * 3)], idx_v)
    pltpu.sync_copy(w_hbm.at[pl.ds(base_row, CPW * 3)], w_v.at[pl.ds(0, CPW * 3)])

    def chunk_body(ch, _):
        idx_slice = idx_v.at[pl.ds(ch * ROWS_PER_CHUNK, ROWS_PER_CHUNK)]
        pltpu.async_copy(feats_hbm.at[idx_slice], rows_v, sem).wait()

        def child_body(lc, _):
            r0 = lc * 3
            woff = ch * ROWS_PER_CHUNK + r0
            wv = w_v[pl.ds(woff, 16)]
            w0 = jnp.full((16,), wv[0], jnp.float32)
            w1 = jnp.full((16,), wv[1], jnp.float32)
            w2 = jnp.full((16,), wv[2], jnp.float32)
            oc = ch * CHUNK + lc
            for dc in range(C // 16):
                sl = pl.ds(dc * 16, 16)
                acc = (w0 * rows_v[r0, sl] + w1 * rows_v[r0 + 1, sl]
                       + w2 * rows_v[r0 + 2, sl])
                out_v[oc, sl] = acc
            return ()

        lax.fori_loop(0, CHUNK, child_body, (), unroll=False)
        return ()

    lax.fori_loop(0, NCHUNK, chunk_body, (), unroll=False)
    pltpu.sync_copy(out_v, out_hbm.at[pl.ds(base_child, CPW)])


def _gather_sc(feats_flat, idx_flat, w_flat):
    mesh = plsc.VectorSubcoreMesh(core_axis_name="c", subcore_axis_name="s",
                                  num_cores=NC, num_subcores=NS)
    f = pl.kernel(
        _gather_body,
        out_type=jax.ShapeDtypeStruct((M, C), jnp.float32),
        mesh=mesh,
        scratch_types=[
            pltpu.VMEM((CPW * 3,), jnp.int32),
            pltpu.VMEM((CPW * 3 + 16,), jnp.float32),
            pltpu.VMEM((ROWS_PER_CHUNK, C), jnp.float32),
            pltpu.VMEM((CPW, C), jnp.float32),
            pltpu.SemaphoreType.DMA,
        ],
    )
    return f(feats_flat, idx_flat, w_flat)


@jax.jit
def kernel(xyz, feats, child_xyz):
    xyz_t = jnp.transpose(xyz, (0, 2, 1))
    feats_flat = jnp.transpose(feats, (0, 2, 1)).reshape(BS * N, C)
    outs = []
    for b in range(BS):
        idx, w = _three_nn_tc(child_xyz, xyz_t, b)
        outs.append(_gather_sc(feats_flat, idx.reshape(-1), w.reshape(-1)))
    out = jnp.stack(outs)
    child_feats = jnp.transpose(out, (0, 2, 1))
    return (child_xyz, child_feats)
